# baseline (speedup 1.0000x reference)
import functools

import jax
import jax.numpy as jnp
from jax.experimental import pallas as pl
from jax.experimental.pallas import tpu as pltpu

_EPS = 1e-5
_GRID = 10
_NODE_ROWS = 10000
_PACK_ROWS = 40000


def _ln_body(n_ref, e_ref, t_ref, ng_ref, nb_ref, s_ref, eg_ref, eb_ref,
             tg_ref, tb_ref, no_ref, eo_ref, to_ref):
    x = n_ref[...]
    mu = jnp.mean(x, axis=-1, keepdims=True)
    xc = x - mu
    var = jnp.mean(xc * xc, axis=-1, keepdims=True)
    no_ref[...] = xc * jax.lax.rsqrt(var + _EPS) * ng_ref[...] + nb_ref[...]

    s = s_ref[...]
    lane = jax.lax.broadcasted_iota(jnp.int32, e_ref.shape, 1)
    feat = (lane % 16) >= 3
    for ref, g_ref, b_ref, o_ref in ((e_ref, eg_ref, eb_ref, eo_ref),
                                     (t_ref, tg_ref, tb_ref, to_ref)):
        x = ref[...]
        s1 = jnp.dot(x, s, preferred_element_type=jnp.float32)
        s2 = jnp.dot(x * x, s, preferred_element_type=jnp.float32)
        mu = s1 * (1.0 / 13.0)
        var = s2 * (1.0 / 13.0) - mu * mu
        y = (x - mu) * jax.lax.rsqrt(var + _EPS) * g_ref[...] + b_ref[...]
        o_ref[...] = jnp.where(feat, y, x)


def _tile128(vec13):
    return jnp.tile(jnp.concatenate([jnp.zeros((3,), vec13.dtype), vec13]), 8)[None, :]


@functools.partial(jax.jit, static_argnums=())
def kernel(node_features, edge_features, triangle_features,
           node_gamma, node_beta, edge_gamma, edge_beta, tri_gamma, tri_beta):
    e2 = edge_features.reshape(_PACK_ROWS, 128)
    t2 = triangle_features.reshape(_PACK_ROWS, 128)

    i = jnp.arange(128)
    same_group = (i[:, None] // 16) == (i[None, :] // 16)
    s_mat = (same_group & ((i[:, None] % 16) >= 3)).astype(jnp.float32)

    nblk = _NODE_ROWS // _GRID
    pblk = _PACK_ROWS // _GRID

    node_out, e_out, t_out = pl.pallas_call(
        _ln_body,
        grid=(_GRID,),
        in_specs=[
            pl.BlockSpec((nblk, 128), lambda i: (i, 0)),
            pl.BlockSpec((pblk, 128), lambda i: (i, 0)),
            pl.BlockSpec((pblk, 128), lambda i: (i, 0)),
            pl.BlockSpec((1, 128), lambda i: (0, 0)),
            pl.BlockSpec((1, 128), lambda i: (0, 0)),
            pl.BlockSpec((128, 128), lambda i: (0, 0)),
            pl.BlockSpec((1, 128), lambda i: (0, 0)),
            pl.BlockSpec((1, 128), lambda i: (0, 0)),
            pl.BlockSpec((1, 128), lambda i: (0, 0)),
            pl.BlockSpec((1, 128), lambda i: (0, 0)),
        ],
        out_specs=[
            pl.BlockSpec((nblk, 128), lambda i: (i, 0)),
            pl.BlockSpec((pblk, 128), lambda i: (i, 0)),
            pl.BlockSpec((pblk, 128), lambda i: (i, 0)),
        ],
        out_shape=[
            jax.ShapeDtypeStruct((_NODE_ROWS, 128), jnp.float32),
            jax.ShapeDtypeStruct((_PACK_ROWS, 128), jnp.float32),
            jax.ShapeDtypeStruct((_PACK_ROWS, 128), jnp.float32),
        ],
        compiler_params=pltpu.CompilerParams(
            dimension_semantics=("arbitrary",)),
    )(node_features, e2, t2,
      node_gamma[None, :], node_beta[None, :], s_mat,
      _tile128(edge_gamma), _tile128(edge_beta),
      _tile128(tri_gamma), _tile128(tri_beta))

    return (node_out, e_out.reshape(320000, 16), t_out.reshape(320000, 16))


# speedup vs baseline: 9.3736x; 9.3736x over previous
import jax
import jax.numpy as jnp
from jax.experimental import pallas as pl
from jax.experimental.pallas import tpu as pltpu

_EPS = 1e-5
_GRID = 10
_NODE_ROWS = 10000
_EDGE_ROWS = 320000
_BL = _EDGE_ROWS // _GRID
_NBLK = _NODE_ROWS // _GRID


def _ln_body(n_ref, e_ref, t_ref, ng_ref, nb_ref, eg_ref, eb_ref,
             tg_ref, tb_ref, no_ref, eo_ref, to_ref):
    x = n_ref[...]
    mu = jnp.mean(x, axis=-1, keepdims=True)
    xc = x - mu
    var = jnp.mean(xc * xc, axis=-1, keepdims=True)
    no_ref[...] = xc * jax.lax.rsqrt(var + _EPS) * ng_ref[...] + nb_ref[...]

    row = jax.lax.broadcasted_iota(jnp.int32, (16, _BL), 0)
    geom = row < 3
    for ref, g_ref, b_ref, o_ref in ((e_ref, eg_ref, eb_ref, eo_ref),
                                     (t_ref, tg_ref, tb_ref, to_ref)):
        x = ref[...]
        xf = x[3:16, :]
        s1 = jnp.sum(xf, axis=0, keepdims=True)
        s2 = jnp.sum(xf * xf, axis=0, keepdims=True)
        mu = s1 * (1.0 / 13.0)
        var = s2 * (1.0 / 13.0) - mu * mu
        y = (x - mu) * jax.lax.rsqrt(var + _EPS) * g_ref[...] + b_ref[...]
        o_ref[...] = jnp.where(geom, x, y)


def _col16(vec13):
    return jnp.concatenate([jnp.zeros((3,), vec13.dtype), vec13])[:, None]


def kernel(node_features, edge_features, triangle_features,
           node_gamma, node_beta, edge_gamma, edge_beta, tri_gamma, tri_beta):
    e_t = edge_features.T
    t_t = triangle_features.T

    node_out, e_out, t_out = pl.pallas_call(
        _ln_body,
        grid=(_GRID,),
        in_specs=[
            pl.BlockSpec((_NBLK, 128), lambda i: (i, 0)),
            pl.BlockSpec((16, _BL), lambda i: (0, i)),
            pl.BlockSpec((16, _BL), lambda i: (0, i)),
            pl.BlockSpec((1, 128), lambda i: (0, 0)),
            pl.BlockSpec((1, 128), lambda i: (0, 0)),
            pl.BlockSpec((16, 1), lambda i: (0, 0)),
            pl.BlockSpec((16, 1), lambda i: (0, 0)),
            pl.BlockSpec((16, 1), lambda i: (0, 0)),
            pl.BlockSpec((16, 1), lambda i: (0, 0)),
        ],
        out_specs=[
            pl.BlockSpec((_NBLK, 128), lambda i: (i, 0)),
            pl.BlockSpec((16, _BL), lambda i: (0, i)),
            pl.BlockSpec((16, _BL), lambda i: (0, i)),
        ],
        out_shape=[
            jax.ShapeDtypeStruct((_NODE_ROWS, 128), jnp.float32),
            jax.ShapeDtypeStruct((16, _EDGE_ROWS), jnp.float32),
            jax.ShapeDtypeStruct((16, _EDGE_ROWS), jnp.float32),
        ],
        compiler_params=pltpu.CompilerParams(
            dimension_semantics=("arbitrary",)),
    )(node_features, e_t, t_t,
      node_gamma[None, :], node_beta[None, :],
      _col16(edge_gamma), _col16(edge_beta),
      _col16(tri_gamma), _col16(tri_beta))

    return (node_out, e_out.T, t_out.T)


# speedup vs baseline: 13.0629x; 1.3936x over previous
import jax
import jax.numpy as jnp
from jax.experimental import pallas as pl
from jax.experimental.pallas import tpu as pltpu

_EPS = 1e-5
_GRID = 10
_NODE_ROWS = 10000
_EDGE_ROWS = 320000
_BL = _EDGE_ROWS // _GRID
_NBLK = _NODE_ROWS // _GRID


def _ln_body(n_ref, e_ref, t_ref, ng_ref, nb_ref, m_ref, eg_ref, eb_ref,
             tg_ref, tb_ref, no_ref, eo_ref, to_ref):
    x = n_ref[...]
    mu = jnp.mean(x, axis=-1, keepdims=True)
    xc = x - mu
    var = jnp.mean(xc * xc, axis=-1, keepdims=True)
    no_ref[...] = xc * jax.lax.rsqrt(var + _EPS) * ng_ref[...] + nb_ref[...]

    row8 = jax.lax.broadcasted_iota(jnp.int32, (8, _BL), 0)
    geom8 = row8 < 3
    m8 = m_ref[...]
    for ref, g_ref, b_ref, o_ref in ((e_ref, eg_ref, eb_ref, eo_ref),
                                     (t_ref, tg_ref, tb_ref, to_ref)):
        x = ref[...]
        s1 = jnp.dot(m8, x, preferred_element_type=jnp.float32)
        s2 = jnp.dot(m8, x * x, preferred_element_type=jnp.float32)
        mu = s1 * (1.0 / 13.0)
        var = s2 * (1.0 / 13.0) - mu * mu
        rstd = jax.lax.rsqrt(var + _EPS)
        g = g_ref[...]
        b = b_ref[...]
        p_lo = rstd * g[0:8, :]
        p_hi = rstd * g[8:16, :]
        q_lo = b[0:8, :] - mu * p_lo
        q_hi = b[8:16, :] - mu * p_hi
        xlo = x[0:8, :]
        xhi = x[8:16, :]
        o_ref[0:8, :] = jnp.where(geom8, xlo, xlo * p_lo + q_lo)
        o_ref[8:16, :] = xhi * p_hi + q_hi


def _col16(vec13):
    return jnp.concatenate([jnp.zeros((3,), vec13.dtype), vec13])[:, None]


def kernel(node_features, edge_features, triangle_features,
           node_gamma, node_beta, edge_gamma, edge_beta, tri_gamma, tri_beta):
    e_t = edge_features.T
    t_t = triangle_features.T
    m8 = (jnp.arange(16)[None, :] >= 3).astype(jnp.float32) * jnp.ones((8, 1), jnp.float32)

    node_out, e_out, t_out = pl.pallas_call(
        _ln_body,
        grid=(_GRID,),
        in_specs=[
            pl.BlockSpec((_NBLK, 128), lambda i: (i, 0)),
            pl.BlockSpec((16, _BL), lambda i: (0, i)),
            pl.BlockSpec((16, _BL), lambda i: (0, i)),
            pl.BlockSpec((1, 128), lambda i: (0, 0)),
            pl.BlockSpec((1, 128), lambda i: (0, 0)),
            pl.BlockSpec((8, 16), lambda i: (0, 0)),
            pl.BlockSpec((16, 1), lambda i: (0, 0)),
            pl.BlockSpec((16, 1), lambda i: (0, 0)),
            pl.BlockSpec((16, 1), lambda i: (0, 0)),
            pl.BlockSpec((16, 1), lambda i: (0, 0)),
        ],
        out_specs=[
            pl.BlockSpec((_NBLK, 128), lambda i: (i, 0)),
            pl.BlockSpec((16, _BL), lambda i: (0, i)),
            pl.BlockSpec((16, _BL), lambda i: (0, i)),
        ],
        out_shape=[
            jax.ShapeDtypeStruct((_NODE_ROWS, 128), jnp.float32),
            jax.ShapeDtypeStruct((16, _EDGE_ROWS), jnp.float32),
            jax.ShapeDtypeStruct((16, _EDGE_ROWS), jnp.float32),
        ],
        compiler_params=pltpu.CompilerParams(
            dimension_semantics=("arbitrary",)),
    )(node_features, e_t, t_t,
      node_gamma[None, :], node_beta[None, :], m8,
      _col16(edge_gamma), _col16(edge_beta),
      _col16(tri_gamma), _col16(tri_beta))

    return (node_out, e_out.T, t_out.T)
